# baseline (speedup 1.0000x reference)
import functools

import jax
import jax.numpy as jnp
from jax import lax
from jax.experimental import pallas as pl
from jax.experimental.pallas import tpu as pltpu
from jax.experimental.pallas import tpu_sc as plsc

B = 4
DIM = 1024
T = 4096
G = 2
CD = 4
DPG = DIM // G
R = 2
GC = G * CD
NSLOT = G * R
HALF_L = (5.0 - 1.0) * (1.0 + 1e-3) / 2.0
EPS = 1e-5
TT = 2048
NT = T // TT
KA = 16
NW = 32
CHUNK = B * NSLOT * T // NW
NBIN = 640


def _gfsq_body(x_ref, win_ref, bin_ref, wout_ref, feat_ref, ind_ref):
    xb = x_ref[0].astype(jnp.bfloat16)
    z = jax.lax.dot(win_ref[...], xb,
                    preferred_element_type=jnp.float32)
    z = z + bin_ref[:, 0:1]

    r0 = jnp.round(jnp.tanh(z) * HALF_L)
    res = z - 0.5 * r0
    r1 = jnp.round(jnp.tanh(res * 4.0) * HALF_L)
    q = 0.5 * r0 + 0.125 * r1

    q_aug = jnp.concatenate(
        [q.astype(jnp.bfloat16),
         jnp.ones((2, TT), jnp.bfloat16),
         jnp.zeros((KA - GC - 2, TT), jnp.bfloat16)], axis=0)
    feat_ref[0] = jax.lax.dot(wout_ref[...], q_aug,
                              preferred_element_type=jnp.float32)

    d0 = (r0 + 2.0).astype(jnp.int32)
    d1 = (r1 + 2.0).astype(jnp.int32)
    rows = []
    for g in range(G):
        for d in (d0, d1):
            lo = d[4 * g:4 * g + 1, :] + 5 * d[4 * g + 1:4 * g + 2, :]
            hi = d[4 * g + 2:4 * g + 3, :] + 5 * d[4 * g + 3:4 * g + 4, :]
            rows.append(lo + 25 * hi)
    pad = jnp.zeros((8 - NSLOT, TT), jnp.int32)
    ind_ref[0] = jnp.concatenate(rows + [pad], axis=0)


def _sc_hist_body(ind_ref, out_ref, chunk_ref, hist_ref):
    wid = lax.axis_index("c") * 16 + lax.axis_index("s")
    k = wid // 8
    rr = wid % 8
    b = rr // 2
    half = rr % 2

    zero16 = jnp.zeros((16,), jnp.float32)
    for c in range(16 * NBIN // 16):
        hist_ref[pl.ds(c * 16, 16)] = zero16

    pltpu.sync_copy(ind_ref.at[b, k, pl.ds(half * CHUNK, CHUNK)], chunk_ref)

    lane_base = lax.broadcasted_iota(jnp.int32, (16,), 0) * NBIN
    ones16 = jnp.ones((16,), jnp.float32)

    def body(i, carry):
        idx16 = chunk_ref[pl.ds(i * 16, 16)]
        plsc.addupdate_scatter(hist_ref, [lane_base + idx16], ones16)
        return carry

    lax.fori_loop(0, CHUNK // 16, body, 0)

    pltpu.sync_copy(hist_ref, out_ref.at[wid])


_sc_hist = functools.partial(
    pl.kernel,
    out_type=jax.ShapeDtypeStruct((NW, 16 * NBIN), jnp.float32),
    mesh=plsc.VectorSubcoreMesh(core_axis_name="c", subcore_axis_name="s"),
    scratch_types=[
        pltpu.VMEM((CHUNK,), jnp.int32),
        pltpu.VMEM((16 * NBIN,), jnp.float32),
    ],
    compiler_params=pltpu.CompilerParams(needs_layout_passes=False),
)(_sc_hist_body)


def _perp_body(cnt_ref, perp_ref):
    for k in range(NSLOT):
        c = jnp.sum(cnt_ref[128 * k:128 * (k + 1), :], axis=0,
                    keepdims=True)
        p0 = c * (1.0 / (B * T))
        s = jnp.sum(p0)
        p = p0 / (s + EPS)
        ent = jnp.sum(p * jnp.log(p + EPS))
        perp_ref[k:k + 1, :] = jnp.broadcast_to(jnp.exp(-ent), (1, 128))
    perp_ref[NSLOT:, :] = jnp.zeros((8 - NSLOT, 128), jnp.float32)


def kernel(x, Win, bin_, Wout, bout):
    f32 = jnp.float32
    bf16 = jnp.bfloat16
    zin = jnp.zeros((CD, DPG), bf16)
    w_in = jnp.concatenate([
        jnp.concatenate([Win[0].astype(bf16), zin], axis=1),
        jnp.concatenate([zin, Win[1].astype(bf16)], axis=1)], axis=0)
    zout = jnp.zeros((DPG, CD), bf16)
    w_out = jnp.concatenate([
        jnp.concatenate([Wout[0].astype(bf16), zout], axis=1),
        jnp.concatenate([zout, Wout[1].astype(bf16)], axis=1)], axis=0)
    bo = bout.reshape(DIM, 1)
    bo_hi = bo.astype(bf16)
    bo_lo = (bo - bo_hi.astype(f32)).astype(bf16)
    w_out = jnp.concatenate(
        [w_out, bo_hi, bo_lo, jnp.zeros((DIM, KA - GC - 2), bf16)], axis=1)
    b_in = jnp.tile(bin_.reshape(GC, 1), (1, 128))

    feat, ind8 = pl.pallas_call(
        _gfsq_body,
        grid=(B, NT),
        in_specs=[
            pl.BlockSpec((1, DIM, TT), lambda b, t: (b, 0, t)),
            pl.BlockSpec((GC, DIM), lambda b, t: (0, 0)),
            pl.BlockSpec((GC, 128), lambda b, t: (0, 0)),
            pl.BlockSpec((DIM, KA), lambda b, t: (0, 0)),
        ],
        out_specs=[
            pl.BlockSpec((1, DIM, TT), lambda b, t: (b, 0, t)),
            pl.BlockSpec((1, 8, TT), lambda b, t: (b, 0, t)),
        ],
        out_shape=[
            jax.ShapeDtypeStruct((B, DIM, T), f32),
            jax.ShapeDtypeStruct((B, 8, T), jnp.int32),
        ],
        compiler_params=pltpu.CompilerParams(
            dimension_semantics=("arbitrary", "arbitrary")),
    )(x, w_in, b_in, w_out)

    counts = _sc_hist(ind8)

    perp = pl.pallas_call(
        _perp_body,
        out_shape=jax.ShapeDtypeStruct((8, 128), f32),
    )(counts.reshape(NW * 16, NBIN))

    perplexity = perp[:NSLOT, 0]
    ind = ind8[:, :NSLOT, :]
    return (jnp.zeros_like(perplexity), feat, perplexity, ind)


# speedup vs baseline: 1.4066x; 1.4066x over previous
import jax
import jax.numpy as jnp
from jax.experimental import pallas as pl
from jax.experimental.pallas import tpu as pltpu

B = 4
DIM = 1024
T = 4096
G = 2
CD = 4
DPG = DIM // G
R = 2
GC = G * CD
NSLOT = G * R
HALF_L = (5.0 - 1.0) * (1.0 + 1e-3) / 2.0
EPS = 1e-5
TT = 2048
NT = T // TT
KA = 16


def _gfsq_body(x_ref, win_ref, bin_ref, wout_ref,
               feat_ref, ind_ref, perp_ref, counts_ref):
    b = pl.program_id(0)
    t = pl.program_id(1)

    @pl.when(jnp.logical_and(b == 0, t == 0))
    def _init():
        counts_ref[...] = jnp.zeros_like(counts_ref)

    xb = x_ref[0].astype(jnp.bfloat16)
    z = jax.lax.dot(win_ref[...], xb,
                    preferred_element_type=jnp.float32)
    z = z + bin_ref[:, 0:1]

    r0 = jnp.round(jnp.tanh(z) * HALF_L)
    res = z - 0.5 * r0
    r1 = jnp.round(jnp.tanh(res * 4.0) * HALF_L)
    q = 0.5 * r0 + 0.125 * r1

    q_aug = jnp.concatenate(
        [q.astype(jnp.bfloat16),
         jnp.ones((2, TT), jnp.bfloat16),
         jnp.zeros((KA - GC - 2, TT), jnp.bfloat16)], axis=0)
    feat_ref[0] = jax.lax.dot(wout_ref[...], q_aug,
                              preferred_element_type=jnp.float32)

    d0 = (r0 + 2.0).astype(jnp.int32)
    d1 = (r1 + 2.0).astype(jnp.int32)
    iota = jax.lax.broadcasted_iota(jnp.int32, (32, 1), 0)
    rows, mls, mhs = [], [], []
    for g in range(G):
        for d in (d0, d1):
            lo = d[4 * g:4 * g + 1, :] + 5 * d[4 * g + 1:4 * g + 2, :]
            hi = d[4 * g + 2:4 * g + 3, :] + 5 * d[4 * g + 3:4 * g + 4, :]
            rows.append(lo + 25 * hi)
            mls.append((iota == lo).astype(jnp.bfloat16))
            mhs.append((iota == hi).astype(jnp.bfloat16))
    ml = jnp.concatenate(mls, axis=0)
    mh = jnp.concatenate(mhs, axis=0)
    c = jax.lax.dot_general(ml, mh, (((1,), (1,)), ((), ())),
                            preferred_element_type=jnp.float32)
    counts_ref[...] = counts_ref[...] + c

    pad = jnp.zeros((8 - NSLOT, TT), jnp.int32)
    ind_ref[0] = jnp.concatenate(rows + [pad], axis=0)

    @pl.when(jnp.logical_and(b == B - 1, t == NT - 1))
    def _fin():
        for k in range(NSLOT):
            cnt = counts_ref[32 * k:32 * k + 32, 32 * k:32 * k + 32]
            p0 = cnt * (1.0 / (B * T))
            s = jnp.sum(p0)
            p = p0 / (s + EPS)
            ent = jnp.sum(p * jnp.log(p + EPS))
            perp_ref[k:k + 1, :] = jnp.broadcast_to(jnp.exp(-ent), (1, 128))


def kernel(x, Win, bin_, Wout, bout):
    f32 = jnp.float32
    bf16 = jnp.bfloat16
    zin = jnp.zeros((CD, DPG), bf16)
    w_in = jnp.concatenate([
        jnp.concatenate([Win[0].astype(bf16), zin], axis=1),
        jnp.concatenate([zin, Win[1].astype(bf16)], axis=1)], axis=0)
    zout = jnp.zeros((DPG, CD), bf16)
    w_out = jnp.concatenate([
        jnp.concatenate([Wout[0].astype(bf16), zout], axis=1),
        jnp.concatenate([zout, Wout[1].astype(bf16)], axis=1)], axis=0)
    bo = bout.reshape(DIM, 1)
    bo_hi = bo.astype(bf16)
    bo_lo = (bo - bo_hi.astype(f32)).astype(bf16)
    w_out = jnp.concatenate(
        [w_out, bo_hi, bo_lo, jnp.zeros((DIM, KA - GC - 2), bf16)], axis=1)
    b_in = jnp.tile(bin_.reshape(GC, 1), (1, 128))

    feat, ind8, perp = pl.pallas_call(
        _gfsq_body,
        grid=(B, NT),
        in_specs=[
            pl.BlockSpec((1, DIM, TT), lambda b, t: (b, 0, t)),
            pl.BlockSpec((GC, DIM), lambda b, t: (0, 0)),
            pl.BlockSpec((GC, 128), lambda b, t: (0, 0)),
            pl.BlockSpec((DIM, KA), lambda b, t: (0, 0)),
        ],
        out_specs=[
            pl.BlockSpec((1, DIM, TT), lambda b, t: (b, 0, t)),
            pl.BlockSpec((1, 8, TT), lambda b, t: (b, 0, t)),
            pl.BlockSpec((8, 128), lambda b, t: (0, 0)),
        ],
        out_shape=[
            jax.ShapeDtypeStruct((B, DIM, T), f32),
            jax.ShapeDtypeStruct((B, 8, T), jnp.int32),
            jax.ShapeDtypeStruct((8, 128), f32),
        ],
        scratch_shapes=[pltpu.VMEM((128, 128), f32)],
        compiler_params=pltpu.CompilerParams(
            dimension_semantics=("arbitrary", "arbitrary")),
    )(x, w_in, b_in, w_out)

    perplexity = perp[:NSLOT, 0]
    ind = ind8[:, :NSLOT, :]
    return (jnp.zeros_like(perplexity), feat, perplexity, ind)
